# baseline (speedup 1.0000x reference)
import jax
import jax.numpy as jnp
from jax import lax
from jax.experimental import pallas as pl
from jax.experimental.pallas import tpu as pltpu

_N = 10000
_D = 128
_BM = 200
_MB = _N // _BM
_NBUF = 4
_FB = 200
_NF = _N // _FB


_SPLIT = 96


def _am_copy_a(am_hbm, buf_ref, sems, block, slot):
    return pltpu.make_async_copy(
        am_hbm.at[pl.ds(block * _BM, _SPLIT), :],
        buf_ref.at[slot, pl.ds(0, _SPLIT), :],
        sems.at[slot],
    )


def _am_copy_b(am_hbm, buf_ref, sems_b, block, slot):
    return pltpu.make_async_copy(
        am_hbm.at[pl.ds(block * _BM + _SPLIT, _BM - _SPLIT), :],
        buf_ref.at[slot, pl.ds(_SPLIT, _BM - _SPLIT), :],
        sems_b.at[slot],
    )


def _out_copy(h_ref, out_hbm, sem_o, chunk):
    rows = pl.ds(chunk * _FB, _FB)
    return pltpu.make_async_copy(h_ref.at[rows, :], out_hbm.at[rows, :], sem_o)


def _fused_body(x_hbm, w_ref, g_ref, be_ref, am_hbm, out_hbm,
                xw_ref, s1_ref, s2_ref, h_ref, buf_ref, sems, sems_b, sem_x, sem_o):
    i = pl.program_id(0)

    @pl.when(i == 0)
    def _init():
        for s in range(_NBUF):
            _am_copy_a(am_hbm, buf_ref, sems, s, s).start()
            _am_copy_b(am_hbm, buf_ref, sems_b, s, s).start()
        x_cp = pltpu.make_async_copy(x_hbm, h_ref.at[pl.ds(0, _N), :], sem_x)
        x_cp.start()
        x_cp.wait()
        for j in range(4):
            rows = pl.ds(j * (_N // 4), _N // 4)
            xw_ref[rows, :] = lax.dot_general(
                h_ref[rows, :], w_ref[...],
                dimension_numbers=(((1,), (1,)), ((), ())),
                precision=lax.Precision.HIGHEST,
                preferred_element_type=jnp.float32,
            )
        s1_ref[...] = jnp.zeros_like(s1_ref)
        s2_ref[...] = jnp.zeros_like(s2_ref)

    slot = lax.rem(i, _NBUF)
    _am_copy_a(am_hbm, buf_ref, sems, i, slot).wait()
    _am_copy_b(am_hbm, buf_ref, sems_b, i, slot).wait()

    h = lax.dot_general(
        buf_ref[slot], xw_ref[...],
        dimension_numbers=(((1,), (0,)), ((), ())),
        precision=lax.Precision.DEFAULT,
        preferred_element_type=jnp.float32,
    )
    h_ref[pl.ds(i * _BM, _BM), :] = h
    s1_ref[...] += jnp.sum(h, axis=0, keepdims=True)
    s2_ref[...] += jnp.sum(h * h, axis=0, keepdims=True)

    @pl.when(i + _NBUF < _MB)
    def _refill():
        _am_copy_a(am_hbm, buf_ref, sems, i + _NBUF, slot).start()
        _am_copy_b(am_hbm, buf_ref, sems_b, i + _NBUF, slot).start()

    @pl.when(i == _MB - 1)
    def _finalize():
        inv_n = jnp.float32(1.0 / _N)
        mean = s1_ref[...] * inv_n
        var = s2_ref[...] * inv_n - mean * mean
        scale = g_ref[...] * lax.rsqrt(var + 1e-5)
        shift = be_ref[...] - mean * scale

        def _norm_chunk(j, carry):
            rows = pl.ds(j * _FB, _FB)
            h_ref[rows, :] = jnp.maximum(h_ref[rows, :] * scale + shift, 0.0)
            _out_copy(h_ref, out_hbm, sem_o, j).start()
            return carry

        lax.fori_loop(0, _NF, _norm_chunk, 0)

        def _drain(j, carry):
            _out_copy(h_ref, out_hbm, sem_o, j).wait()
            return carry

        lax.fori_loop(0, _NF, _drain, 0)


def kernel(x, am, W, b, gamma, beta):
    del b
    g2 = gamma.reshape(1, _D)
    be2 = beta.reshape(1, _D)
    return pl.pallas_call(
        _fused_body,
        grid=(_MB,),
        in_specs=[
            pl.BlockSpec(memory_space=pl.ANY),
            pl.BlockSpec((_D, _D), lambda i: (0, 0)),
            pl.BlockSpec((1, _D), lambda i: (0, 0)),
            pl.BlockSpec((1, _D), lambda i: (0, 0)),
            pl.BlockSpec(memory_space=pl.ANY),
        ],
        out_specs=pl.BlockSpec(memory_space=pl.ANY),
        out_shape=jax.ShapeDtypeStruct((_N, _D), jnp.float32),
        scratch_shapes=[
            pltpu.VMEM((_N, _D), jnp.float32),
            pltpu.VMEM((1, _D), jnp.float32),
            pltpu.VMEM((1, _D), jnp.float32),
            pltpu.VMEM((_N, _D), jnp.float32),
            pltpu.VMEM((_NBUF, _BM, _N), jnp.float32),
            pltpu.SemaphoreType.DMA((_NBUF,)),
            pltpu.SemaphoreType.DMA((_NBUF,)),
            pltpu.SemaphoreType.DMA,
            pltpu.SemaphoreType.DMA,
        ],
    )(x, W, g2, be2, am)


# speedup vs baseline: 1.0622x; 1.0622x over previous
import jax
import jax.numpy as jnp
from jax import lax
from jax.experimental import pallas as pl
from jax.experimental.pallas import tpu as pltpu

_N = 10000
_D = 128
_BM = 200
_MB = _N // _BM
_NBUF = 3
_FB = 500
_NF = _N // _FB


_SPLIT = 96


def _am_copy_a(am_hbm, buf_ref, sems, block, slot):
    return pltpu.make_async_copy(
        am_hbm.at[pl.ds(block * _BM, _SPLIT), :],
        buf_ref.at[slot, pl.ds(0, _SPLIT), :],
        sems.at[slot],
    )


def _am_copy_b(am_hbm, buf_ref, sems_b, block, slot):
    return pltpu.make_async_copy(
        am_hbm.at[pl.ds(block * _BM + _SPLIT, _BM - _SPLIT), :],
        buf_ref.at[slot, pl.ds(_SPLIT, _BM - _SPLIT), :],
        sems_b.at[slot],
    )


def _out_copy(h_ref, out_hbm, sem_o, chunk):
    rows = pl.ds(chunk * _FB, _FB)
    return pltpu.make_async_copy(h_ref.at[rows, :], out_hbm.at[rows, :], sem_o)


def _fused_body(x_hbm, w_ref, g_ref, be_ref, am_hbm, out_hbm,
                xw_ref, s1_ref, s2_ref, h_ref, buf_ref, sems, sems_b, sem_x, sem_o):
    i = pl.program_id(0)

    @pl.when(i == 0)
    def _init():
        for s in range(_NBUF):
            _am_copy_a(am_hbm, buf_ref, sems, s, s).start()
            _am_copy_b(am_hbm, buf_ref, sems_b, s, s).start()
        x_cp = pltpu.make_async_copy(x_hbm, h_ref.at[pl.ds(0, _N), :], sem_x)
        x_cp.start()
        x_cp.wait()
        xw_ref[...] = lax.dot_general(
            h_ref[...], w_ref[...],
            dimension_numbers=(((1,), (1,)), ((), ())),
            precision=lax.Precision.HIGHEST,
            preferred_element_type=jnp.float32,
        )
        s1_ref[...] = jnp.zeros_like(s1_ref)
        s2_ref[...] = jnp.zeros_like(s2_ref)

    slot = lax.rem(i, _NBUF)
    _am_copy_a(am_hbm, buf_ref, sems, i, slot).wait()
    _am_copy_b(am_hbm, buf_ref, sems_b, i, slot).wait()

    h = lax.dot_general(
        buf_ref[slot], xw_ref[...],
        dimension_numbers=(((1,), (0,)), ((), ())),
        precision=lax.Precision.DEFAULT,
        preferred_element_type=jnp.float32,
    )
    h_ref[pl.ds(i * _BM, _BM), :] = h
    s1_ref[...] += jnp.sum(h, axis=0, keepdims=True)
    s2_ref[...] += jnp.sum(h * h, axis=0, keepdims=True)

    @pl.when(i + _NBUF < _MB)
    def _refill():
        _am_copy_a(am_hbm, buf_ref, sems, i + _NBUF, slot).start()
        _am_copy_b(am_hbm, buf_ref, sems_b, i + _NBUF, slot).start()

    @pl.when(i == _MB - 1)
    def _finalize():
        inv_n = jnp.float32(1.0 / _N)
        mean = s1_ref[...] * inv_n
        var = s2_ref[...] * inv_n - mean * mean
        scale = g_ref[...] * lax.rsqrt(var + 1e-5)
        shift = be_ref[...] - mean * scale

        def _norm_chunk(j, carry):
            rows = pl.ds(j * _FB, _FB)
            h_ref[rows, :] = jnp.maximum(h_ref[rows, :] * scale + shift, 0.0)
            _out_copy(h_ref, out_hbm, sem_o, j).start()
            return carry

        lax.fori_loop(0, _NF, _norm_chunk, 0)

        def _drain(j, carry):
            _out_copy(h_ref, out_hbm, sem_o, j).wait()
            return carry

        lax.fori_loop(0, _NF, _drain, 0)


def kernel(x, am, W, b, gamma, beta):
    del b
    g2 = gamma.reshape(1, _D)
    be2 = beta.reshape(1, _D)
    return pl.pallas_call(
        _fused_body,
        grid=(_MB,),
        in_specs=[
            pl.BlockSpec(memory_space=pl.ANY),
            pl.BlockSpec((_D, _D), lambda i: (0, 0)),
            pl.BlockSpec((1, _D), lambda i: (0, 0)),
            pl.BlockSpec((1, _D), lambda i: (0, 0)),
            pl.BlockSpec(memory_space=pl.ANY),
        ],
        out_specs=pl.BlockSpec(memory_space=pl.ANY),
        out_shape=jax.ShapeDtypeStruct((_N, _D), jnp.float32),
        scratch_shapes=[
            pltpu.VMEM((_N, _D), jnp.float32),
            pltpu.VMEM((1, _D), jnp.float32),
            pltpu.VMEM((1, _D), jnp.float32),
            pltpu.VMEM((_N, _D), jnp.float32),
            pltpu.VMEM((_NBUF, _BM, _N), jnp.float32),
            pltpu.SemaphoreType.DMA((_NBUF,)),
            pltpu.SemaphoreType.DMA((_NBUF,)),
            pltpu.SemaphoreType.DMA,
            pltpu.SemaphoreType.DMA,
        ],
    )(x, W, g2, be2, am)
